# baseline (speedup 1.0000x reference)
import functools
import math

import jax
import jax.numpy as jnp
from jax import lax
from jax.experimental import pallas as pl
from jax.experimental.pallas import tpu as pltpu
from jax.experimental.pallas import tpu_sc as plsc

_TRASH = 3.0e8


def _lane_cumsum(x):
    ck = x.shape[-1]
    it = lax.broadcasted_iota(jnp.int32, x.shape, 1)
    y = x
    sh = 1
    while sh < ck:
        y = y + jnp.where(it >= sh, pltpu.roll(y, sh, 1), 0.0)
        sh *= 2
    return y


def _stream_kernel(cls_ref, reg_ref, mask_ref, map_ref,
                   scal_ref, ce_ref,
                   lp_ref, tp_ref, rp_ref, bp_ref,
                   lg_ref, tg_ref, rg_ref, bg_ref,
                   rank_ref, acc):
    b = pl.program_id(0)
    c = pl.program_id(1)
    nprog1 = pl.num_programs(1)
    g = b * nprog1 + c
    last = pl.num_programs(0) * nprog1 - 1

    @pl.when(g == 0)
    def _init():
        for i in range(9):
            acc[i] = 0.0

    cls_b = cls_ref[0]
    msk = mask_ref[0]
    reg_b = reg_ref[0]
    map_b = map_ref[0]

    tr_m = msk[0:1]
    tcl_m = msk[1:2]
    train_m = msk[2:3]

    def ce2(a, bb, t):
        m = jnp.maximum(a, bb)
        mn = jnp.minimum(a, bb)
        logz = m + jnp.log1p(jnp.exp(mn - m))
        return logz - jnp.where(t > 0.5, bb, a)

    ce_tr = ce2(cls_b[0:1], cls_b[1:2], tr_m)
    ce_tcl = ce2(cls_b[2:3], cls_b[3:4], tcl_m)

    pos = tr_m * train_m
    negm = (1.0 - tr_m) * train_m

    ce_ref[0] = jnp.where(negm > 0.5, ce_tr, -1.0)

    xp = reg_b[0:10]
    yp = reg_b[10:20]
    xg = map_b[0:10]
    yg = map_b[10:20]

    lp_ref[0] = jnp.min(xp, axis=0, keepdims=True)
    rp_ref[0] = jnp.max(xp, axis=0, keepdims=True)
    tp_ref[0] = jnp.min(yp[0:5], axis=0, keepdims=True)
    bp_ref[0] = jnp.max(yp[5:10], axis=0, keepdims=True)
    lg_ref[0] = jnp.min(xg, axis=0, keepdims=True)
    rg_ref[0] = jnp.max(xg, axis=0, keepdims=True)
    tg_ref[0] = jnp.min(yg[0:5], axis=0, keepdims=True)
    bg_ref[0] = jnp.max(yg[5:10], axis=0, keepdims=True)

    cum = _lane_cumsum(pos)
    rank_f = acc[8] + cum - pos
    rank_ref[0] = jnp.where(pos > 0.5, rank_f, _TRASH).astype(jnp.int32)

    w = (tr_m + tcl_m) * 0.5
    dx = jnp.abs(xg - xp)
    slx = jnp.sum(jnp.where(dx < 1.0, 0.5 * dx * dx, dx - 0.5), axis=0,
                  keepdims=True)
    dy = jnp.abs(yg - yp)
    sly = jnp.sum(jnp.where(dy < 1.0, 0.5 * dy * dy, dy - 0.5), axis=0,
                  keepdims=True)

    npos_c = jnp.sum(pos)
    acc[0] = acc[0] + npos_c
    acc[1] = acc[1] + jnp.sum(negm)
    acc[2] = acc[2] + jnp.sum(pos * ce_tr)
    acc[3] = acc[3] + jnp.sum(negm * ce_tr)
    acc[4] = acc[4] + jnp.sum(pos * ce_tcl)
    acc[5] = acc[5] + jnp.sum((1.0 - pos) * ce_tcl)
    acc[6] = acc[6] + jnp.sum(pos * w * slx)
    acc[7] = acc[7] + jnp.sum(pos * w * sly)
    acc[8] = acc[8] + npos_c

    @pl.when(g == last)
    def _fin():
        vi = lax.broadcasted_iota(jnp.int32, (1, 128), 1)
        v = jnp.zeros((1, 128), jnp.float32)
        for i in range(8):
            v = jnp.where(vi == i, acc[i], v)
        scal_ref[...] = v


def _tc_stream(cls_l, reg_l, mask_l, map_l, ck):
    bsz, _, s = cls_l.shape
    nb = bsz * (s // ck)
    blk = lambda ch: pl.BlockSpec((1, ch, ck), lambda b, c: (b, 0, c))
    oblk = pl.BlockSpec((1, 1, ck),
                        lambda b, c, _s=(s // ck): (b * _s + c, 0, 0))
    arr = jax.ShapeDtypeStruct((nb, 1, ck), jnp.float32)
    outs = pl.pallas_call(
        _stream_kernel,
        grid=(bsz, s // ck),
        in_specs=[blk(4), blk(20), blk(3), blk(20)],
        out_specs=[pl.BlockSpec((1, 128), lambda b, c: (0, 0))] + [oblk] * 10,
        out_shape=[jax.ShapeDtypeStruct((1, 128), jnp.float32)] + [arr] * 9
                  + [jax.ShapeDtypeStruct((nb, 1, ck), jnp.int32)],
        scratch_shapes=[pltpu.SMEM((16,), jnp.float32)],
    )(cls_l, reg_l, mask_l, map_l)
    return outs


def _sc_scatter_body(ntot, ch, rank_hbm, v0, v1, v2, v3, v4, v5, v6, v7,
                     nn_hbm, bp_hbm, bg_hbm,
                     nn_v, rk_v, d0, d1, d2, d3, d4, d5, d6, d7,
                     i0, i1, i2, i3, sem):
    dv = (d0, d1, d2, d3, d4, d5, d6, d7)
    iv = (i0, i1, i2, i3)
    vh = (v0, v1, v2, v3, v4, v5, v6, v7)
    per_w = ntot // 32
    wid = lax.axis_index("s") * 2 + lax.axis_index("c")
    base = wid * per_w
    pltpu.sync_copy(nn_hbm, nn_v)
    nn = nn_v[...]

    def chunk(ci, carry):
        off = base + ci * ch
        pltpu.sync_copy(rank_hbm.at[pl.ds(off, ch)], rk_v)
        for a in range(8):
            pltpu.sync_copy(vh[a].at[pl.ds(off, ch)], dv[a])
        for j in range(ch // 16):
            r = rk_v[pl.ds(j * 16, 16)]
            for k in range(4):
                q = r + nn * k
                idx = (q & 3) * ntot + (q >> 2)
                idx = jnp.minimum(idx, 4 * ntot)
                iv[k][pl.ds(j * 16, 16)] = idx
        cps = [pltpu.make_async_copy(dv[k], bp_hbm.at[iv[k]], sem)
               for k in range(4)]
        cps += [pltpu.make_async_copy(dv[4 + k], bg_hbm.at[iv[k]], sem)
                for k in range(4)]
        for cp in cps:
            cp.start()
        for cp in cps:
            cp.wait()
        return carry

    lax.fori_loop(0, per_w // ch, chunk, 0)


def _sc_scatter(rank_flat, vals, nn_vec, ntot):
    ch = 128 if (ntot // 32) % 128 == 0 else 80
    mesh = plsc.VectorSubcoreMesh(core_axis_name="c", subcore_axis_name="s")
    obuf = jax.ShapeDtypeStruct((4 * ntot + 8,), jnp.float32)
    fn = pl.kernel(
        functools.partial(_sc_scatter_body, ntot, ch),
        mesh=mesh,
        out_type=[obuf, obuf],
        scratch_types=[pltpu.VMEM((16,), jnp.int32),
                       pltpu.VMEM((ch,), jnp.int32)]
                      + [pltpu.VMEM((ch,), jnp.float32)] * 8
                      + [pltpu.VMEM((ch,), jnp.int32)] * 4
                      + [pltpu.SemaphoreType.DMA],
    )
    return fn(rank_flat, *vals, nn_vec)


def _atan(x):
    t = jnp.abs(x)
    inv = t > 1.0
    z = jnp.where(inv, 1.0 / jnp.maximum(t, 1e-30), t)
    z2 = z * z
    p = jnp.float32(-0.0117212)
    p = p * z2 + 0.05265332
    p = p * z2 + -0.11643287
    p = p * z2 + 0.19354346
    p = p * z2 + -0.33262347
    p = p * z2 + 0.99997726
    p = p * z
    r = jnp.where(inv, jnp.float32(math.pi / 2) - p, p)
    return jnp.where(x < 0.0, -r, r)


def _ciou_block(P, G):
    eps = 1e-6
    px1, py1, px2, py2 = P[0:1], P[1:2], P[2:3], P[3:4]
    gx1, gy1, gx2, gy2 = G[0:1], G[1:2], G[2:3], G[3:4]
    wo = jnp.clip(jnp.minimum(px2, gx2) - jnp.maximum(px1, gx1), 0.0, None)
    ho = jnp.clip(jnp.minimum(py2, gy2) - jnp.maximum(py1, gy1), 0.0, None)
    overlap = wo * ho
    ap = (px2 - px1) * (py2 - py1)
    ag = (gx2 - gx1) * (gy2 - gy1)
    union = ap + ag - overlap + eps
    ious = overlap / union
    cw = jnp.clip(jnp.maximum(px2, gx2) - jnp.minimum(px1, gx1), 0.0, None)
    chh = jnp.clip(jnp.maximum(py2, gy2) - jnp.minimum(py1, gy1), 0.0, None)
    c2 = cw * cw + chh * chh + eps
    rho2 = ((gx1 + gx2) - (px1 + px2)) ** 2 / 4.0 \
        + ((gy1 + gy2) - (py1 + py2)) ** 2 / 4.0
    w1 = px2 - px1
    h1 = py2 - py1 + eps
    w2 = gx2 - gx1
    h2 = gy2 - gy1 + eps
    fct = 4.0 / (math.pi ** 2)
    v = fct * (_atan(w2 / h2) - _atan(w1 / h1)) ** 2
    alpha = (ious > 0.5).astype(jnp.float32) * v / (1.0 - ious + v)
    cious = ious - (rho2 / c2 + alpha * v)
    return 1.0 - jnp.clip(cious, -1.0, 1.0)


def _finish_kernel(ntot, cc, scal_ref, ce_ref, bp_ref, bg_ref, out_ref):
    np_ = scal_ref[0, 0]
    negc = scal_ref[0, 1]
    lpos = scal_ref[0, 2]
    fneg = scal_ref[0, 3]
    tclp = scal_ref[0, 4]
    tcln = scal_ref[0, 5]
    rgx = scal_ref[0, 6]
    rgy = scal_ref[0, 7]
    nn_i = np_.astype(jnp.int32)

    def cbody(i, s):
        sl = pl.ds(i * cc, cc)
        cio = _ciou_block(bp_ref[:, sl], bg_ref[:, sl])
        colid = lax.broadcasted_iota(jnp.int32, (1, cc), 1) + i * cc
        return s + jnp.sum(jnp.where(colid < nn_i, cio, 0.0))

    ciou_sum = lax.fori_loop(0, ntot // cc, cbody, jnp.float32(0.0))

    ce = ce_ref[...]
    n_neg_pos = jnp.minimum(negc, jnp.floor(3.0 * np_))
    kk = jnp.where(np_ > 0, n_neg_pos, jnp.minimum(negc, 100.0))

    def bbody(i, lohi):
        lo, hi = lohi
        mid = lo + (hi - lo) // 2
        tv = lax.bitcast_convert_type(jnp.full((1, 128), mid, jnp.int32),
                                      jnp.float32)
        t = jnp.max(tv)
        cnt = jnp.sum((ce >= t).astype(jnp.float32))
        ok = cnt >= kk
        return (jnp.where(ok, mid, lo), jnp.where(ok, hi, mid))

    lo, _hi = lax.fori_loop(0, 31, bbody,
                            (jnp.int32(0), jnp.int32(0x7F800000)))
    tv = lax.bitcast_convert_type(jnp.full((1, 128), lo, jnp.int32),
                                  jnp.float32)
    t = jnp.max(tv)
    cnt_gt = jnp.sum((ce > t).astype(jnp.float32))
    sum_gt = jnp.sum(jnp.where(ce > t, ce, 0.0))
    topk = jnp.where(kk >= 1.0, sum_gt + (kk - cnt_gt) * t, 0.0)
    loss_neg = jnp.where(kk >= negc, fneg, topk)
    nneg_div = jnp.where(np_ > 0, n_neg_pos, 100.0)
    loss_tr = (lpos + loss_neg) / (np_ + nneg_div)

    has_pos = np_ > 0
    sp = jnp.maximum(np_, 1.0)
    loss_tcl = jnp.where(
        has_pos, tclp / sp + 0.5 * tcln / jnp.maximum(ntot - np_, 1.0), 0.0)
    loss_rx = jnp.where(has_pos, rgx / (sp * 10.0), 0.0)
    loss_ry = jnp.where(has_pos, rgy / (sp * 10.0), 0.0)
    loss_bbox = jnp.where(has_pos, ciou_sum / sp, 0.0)

    vi = lax.broadcasted_iota(jnp.int32, (1, 128), 1)
    v = jnp.zeros((1, 128), jnp.float32)
    for i, val in enumerate([loss_tr, loss_tcl, loss_rx, loss_ry, loss_bbox]):
        v = jnp.where(vi == i, val, v)
    out_ref[...] = v


def _tc_finish(scal, ce2d, bp2d, bg2d, ntot):
    cc = 1280
    nr = ntot // 128
    return pl.pallas_call(
        functools.partial(_finish_kernel, ntot, cc),
        grid=(1,),
        in_specs=[pl.BlockSpec(memory_space=pltpu.SMEM),
                  pl.BlockSpec((nr, 128), lambda i: (0, 0)),
                  pl.BlockSpec((4, ntot), lambda i: (0, 0)),
                  pl.BlockSpec((4, ntot), lambda i: (0, 0))],
        out_specs=pl.BlockSpec((1, 128), lambda i: (0, 0)),
        out_shape=jax.ShapeDtypeStruct((1, 128), jnp.float32),
    )(scal, ce2d, bp2d, bg2d)


def _level(cls4d, reg4d, mask4d, map4d):
    bsz, _, h, w = cls4d.shape
    s = h * w
    n = bsz * s
    ck = 3200 if s % 3200 == 0 else s
    outs = _tc_stream(cls4d.reshape(bsz, 4, s), reg4d.reshape(bsz, 20, s),
                      mask4d.reshape(bsz, 3, s), map4d.reshape(bsz, 20, s),
                      ck)
    scal = outs[0]
    ce = outs[1].reshape(n)
    vals = [o.reshape(n) for o in outs[2:10]]
    rank = outs[10].reshape(n)
    nn_vec = jnp.full((16,), scal[0, 0].astype(jnp.int32), jnp.int32)
    bp, bg = _sc_scatter(rank, vals, nn_vec, n)
    res = _tc_finish(scal, ce.reshape(n // 128, 128),
                     bp[:4 * n].reshape(4, n), bg[:4 * n].reshape(4, n), n)
    return res[0, :5]


def kernel(cls3, reg3, cls4, reg4, cls5, reg5,
           mask3, map3, mask4, map4, mask5, map5):
    tot = jnp.zeros((5,), jnp.float32)
    for (c, r, m, mp) in [(cls3, reg3, mask3, map3),
                          (cls4, reg4, mask4, map4),
                          (cls5, reg5, mask5, map5)]:
        tot = tot + _level(c, r, m, mp)
    return (tot[0], tot[1], tot[2], tot[3], tot[4])


# speedup vs baseline: 324.3766x; 324.3766x over previous
import functools
import math

import jax
import jax.numpy as jnp
from jax import lax
from jax.experimental import pallas as pl
from jax.experimental.pallas import tpu as pltpu
from jax.experimental.pallas import tpu_sc as plsc

_TRASH = 3.0e8


def _lane_cumsum(x):
    ck = x.shape[-1]
    it = lax.broadcasted_iota(jnp.int32, x.shape, 1)
    y = x
    sh = 1
    while sh < ck:
        y = y + jnp.where(it >= sh, pltpu.roll(y, sh, 1), 0.0)
        sh *= 2
    return y


def _stream_kernel(cls_ref, reg_ref, mask_ref, map_ref,
                   scal_ref, ce_ref, vals_ref, rank_ref, acc):
    b = pl.program_id(0)
    c = pl.program_id(1)
    nprog1 = pl.num_programs(1)
    g = b * nprog1 + c
    last = pl.num_programs(0) * nprog1 - 1

    @pl.when(g == 0)
    def _init():
        for i in range(9):
            acc[i] = 0.0

    cls_b = cls_ref[0]
    msk = mask_ref[0]
    reg_b = reg_ref[0]
    map_b = map_ref[0]

    tr_m = msk[0:1]
    tcl_m = msk[1:2]
    train_m = msk[2:3]

    def ce2(a, bb, t):
        m = jnp.maximum(a, bb)
        mn = jnp.minimum(a, bb)
        logz = m + jnp.log1p(jnp.exp(mn - m))
        return logz - jnp.where(t > 0.5, bb, a)

    ce_tr = ce2(cls_b[0:1], cls_b[1:2], tr_m)
    ce_tcl = ce2(cls_b[2:3], cls_b[3:4], tcl_m)

    pos = tr_m * train_m
    negm = (1.0 - tr_m) * train_m

    ce_ref[0] = jnp.where(negm > 0.5, ce_tr, -1.0)

    xp = reg_b[0:10]
    yp = reg_b[10:20]
    xg = map_b[0:10]
    yg = map_b[10:20]

    vals_ref[0] = jnp.concatenate(
        [jnp.min(xp, axis=0, keepdims=True),
         jnp.min(yp[0:5], axis=0, keepdims=True),
         jnp.max(xp, axis=0, keepdims=True),
         jnp.max(yp[5:10], axis=0, keepdims=True),
         jnp.min(xg, axis=0, keepdims=True),
         jnp.min(yg[0:5], axis=0, keepdims=True),
         jnp.max(xg, axis=0, keepdims=True),
         jnp.max(yg[5:10], axis=0, keepdims=True)], axis=0)

    cum = _lane_cumsum(pos)
    rank_f = acc[8] + cum - pos
    rank_ref[0] = jnp.where(pos > 0.5, rank_f, _TRASH).astype(jnp.int32)

    w = (tr_m + tcl_m) * 0.5
    dx = jnp.abs(xg - xp)
    slx = jnp.sum(jnp.where(dx < 1.0, 0.5 * dx * dx, dx - 0.5), axis=0,
                  keepdims=True)
    dy = jnp.abs(yg - yp)
    sly = jnp.sum(jnp.where(dy < 1.0, 0.5 * dy * dy, dy - 0.5), axis=0,
                  keepdims=True)

    npos_c = jnp.sum(pos)
    acc[0] = acc[0] + npos_c
    acc[1] = acc[1] + jnp.sum(negm)
    acc[2] = acc[2] + jnp.sum(pos * ce_tr)
    acc[3] = acc[3] + jnp.sum(negm * ce_tr)
    acc[4] = acc[4] + jnp.sum(pos * ce_tcl)
    acc[5] = acc[5] + jnp.sum((1.0 - pos) * ce_tcl)
    acc[6] = acc[6] + jnp.sum(pos * w * slx)
    acc[7] = acc[7] + jnp.sum(pos * w * sly)
    acc[8] = acc[8] + npos_c

    @pl.when(g == last)
    def _fin():
        vi = lax.broadcasted_iota(jnp.int32, (1, 128), 1)
        v = jnp.zeros((1, 128), jnp.float32)
        for i in range(8):
            v = jnp.where(vi == i, acc[i], v)
        scal_ref[...] = v


def _tc_stream(cls_l, reg_l, mask_l, map_l, ck):
    bsz, _, s = cls_l.shape
    nb = bsz * (s // ck)
    blk = lambda ch: pl.BlockSpec((1, ch, ck), lambda b, c: (b, 0, c))
    imap = lambda b, c, _s=(s // ck): (b * _s + c, 0, 0)
    outs = pl.pallas_call(
        _stream_kernel,
        grid=(bsz, s // ck),
        in_specs=[blk(4), blk(20), blk(3), blk(20)],
        out_specs=[pl.BlockSpec((1, 128), lambda b, c: (0, 0)),
                   pl.BlockSpec((1, 1, ck), imap),
                   pl.BlockSpec((1, 8, ck), imap),
                   pl.BlockSpec((1, 1, ck), imap)],
        out_shape=[jax.ShapeDtypeStruct((1, 128), jnp.float32),
                   jax.ShapeDtypeStruct((nb, 1, ck), jnp.float32),
                   jax.ShapeDtypeStruct((nb, 8, ck), jnp.float32),
                   jax.ShapeDtypeStruct((nb, 1, ck), jnp.int32)],
        scratch_shapes=[pltpu.SMEM((16,), jnp.float32)],
    )(cls_l, reg_l, mask_l, map_l)
    return outs


def _sc_scatter_body(ntot, ck2, wsz, wpg,
                     rank_hbm, vals_hbm, nn_hbm, out_hbm,
                     nn_v, rk_v, v0, v1, v2, v3, i0, i1, i2, i3,
                     zbuf, spm):
    vv = (v0, v1, v2, v3)
    iv = (i0, i1, i2, i3)
    wn = ck2 // wsz
    gn = wn // wpg
    half = 4 * ntot + 2048
    osz = half // 16
    nfull = osz // 2048
    tail = (osz % 2048) // 128
    core = lax.axis_index("c")
    sid = lax.axis_index("s")

    z16 = jnp.zeros((16,), jnp.float32)
    for j in range(2048 // 16):
        zbuf[pl.ds(j * 16, 16)] = z16
    zbase = sid * osz

    def zc(i, c):
        pltpu.sync_copy(zbuf, spm.at[pl.ds(zbase + i * 2048, 2048)])
        return c

    lax.fori_loop(0, nfull, zc, 0)
    for t in range(tail):
        pltpu.sync_copy(
            zbuf.at[pl.ds(0, 128)],
            spm.at[pl.ds(zbase + nfull * 2048 + t * 128, 128)])

    pltpu.sync_copy(nn_hbm, nn_v)
    nn = nn_v[...]
    nt_v = jnp.full((16,), ntot, jnp.int32)
    lane = lax.iota(jnp.int32, 16)
    plsc.subcore_barrier()

    per_w = ntot // 16
    base = sid * per_w

    def chunk(ci, carry):
        off = base + ci * ck2
        pltpu.sync_copy(rank_hbm.at[pl.ds(off, ck2)], rk_v)
        for k in range(4):
            pltpu.sync_copy(
                vals_hbm.at[pl.ds((core * 4 + k) * ntot + off, ck2)], vv[k])

        def group(gi, carry2):
            for wl in range(wpg):
                w = gi * wpg + wl
                for j in range(wsz // 16):
                    r = rk_v[pl.ds(w * wsz + j * 16, 16)]
                    for k in range(4):
                        q = r + nn * k
                        idx = (q & 3) * ntot + (q >> 2)
                        trash = (4 * ntot + k * 128 + j * 16) + lane
                        iv[k][w, pl.ds(j * 16, 16)] = \
                            jnp.where(r < nt_v, idx, trash)
            for wl in range(wpg):
                w = gi * wpg + wl
                sl = pl.ds(w * wsz, wsz)
                for k in range(4):
                    pltpu.sync_copy(vv[k].at[sl], spm.at[iv[k].at[w]],
                                    add=True)
            return carry2

        lax.fori_loop(0, gn, group, 0)
        return carry

    lax.fori_loop(0, per_w // ck2, chunk, 0)

    plsc.subcore_barrier()
    pltpu.sync_copy(spm.at[pl.ds(sid * osz, osz)],
                    out_hbm.at[pl.ds(core * half + sid * osz, osz)])


def _sc_scatter(rank_flat, vals_flat, nn_vec, ntot, ck):
    per_w = ntot // 16
    ck2 = min(ck, per_w)
    wsz = 128 if ck2 % 128 == 0 else 80
    wn = ck2 // wsz
    wpg = 5
    mesh = plsc.VectorSubcoreMesh(core_axis_name="c", subcore_axis_name="s")
    fn = pl.kernel(
        functools.partial(_sc_scatter_body, ntot, ck2, wsz, wpg),
        mesh=mesh,
        out_type=[jax.ShapeDtypeStruct((2 * (4 * ntot + 2048),), jnp.float32)],
        scratch_types=[pltpu.VMEM((16,), jnp.int32),
                       pltpu.VMEM((ck2,), jnp.int32)]
                      + [pltpu.VMEM((ck2,), jnp.float32)] * 4
                      + [pltpu.VMEM((wn, wsz), jnp.int32)] * 4
                      + [pltpu.VMEM((2048,), jnp.float32),
                         pltpu.VMEM_SHARED((4 * ntot + 2048,), jnp.float32)],
    )
    return fn(rank_flat, vals_flat, nn_vec)


def _atan(x):
    t = jnp.abs(x)
    inv = t > 1.0
    z = jnp.where(inv, 1.0 / jnp.maximum(t, 1e-30), t)
    z2 = z * z
    p = jnp.float32(-0.0117212)
    p = p * z2 + 0.05265332
    p = p * z2 + -0.11643287
    p = p * z2 + 0.19354346
    p = p * z2 + -0.33262347
    p = p * z2 + 0.99997726
    p = p * z
    r = jnp.where(inv, jnp.float32(math.pi / 2) - p, p)
    return jnp.where(x < 0.0, -r, r)


def _ciou_block(P, G):
    eps = 1e-6
    px1, py1, px2, py2 = P[0:1], P[1:2], P[2:3], P[3:4]
    gx1, gy1, gx2, gy2 = G[0:1], G[1:2], G[2:3], G[3:4]
    wo = jnp.clip(jnp.minimum(px2, gx2) - jnp.maximum(px1, gx1), 0.0, None)
    ho = jnp.clip(jnp.minimum(py2, gy2) - jnp.maximum(py1, gy1), 0.0, None)
    overlap = wo * ho
    ap = (px2 - px1) * (py2 - py1)
    ag = (gx2 - gx1) * (gy2 - gy1)
    union = ap + ag - overlap + eps
    ious = overlap / union
    cw = jnp.clip(jnp.maximum(px2, gx2) - jnp.minimum(px1, gx1), 0.0, None)
    chh = jnp.clip(jnp.maximum(py2, gy2) - jnp.minimum(py1, gy1), 0.0, None)
    c2 = cw * cw + chh * chh + eps
    rho2 = ((gx1 + gx2) - (px1 + px2)) ** 2 / 4.0 \
        + ((gy1 + gy2) - (py1 + py2)) ** 2 / 4.0
    w1 = px2 - px1
    h1 = py2 - py1 + eps
    w2 = gx2 - gx1
    h2 = gy2 - gy1 + eps
    fct = 4.0 / (math.pi ** 2)
    v = fct * (_atan(w2 / h2) - _atan(w1 / h1)) ** 2
    alpha = (ious > 0.5).astype(jnp.float32) * v / (1.0 - ious + v)
    cious = ious - (rho2 / c2 + alpha * v)
    return 1.0 - jnp.clip(cious, -1.0, 1.0)


def _finish_kernel(ntot, cc, scal_ref, ce_ref, bp_ref, bg_ref, out_ref):
    np_ = scal_ref[0, 0]
    negc = scal_ref[0, 1]
    lpos = scal_ref[0, 2]
    fneg = scal_ref[0, 3]
    tclp = scal_ref[0, 4]
    tcln = scal_ref[0, 5]
    rgx = scal_ref[0, 6]
    rgy = scal_ref[0, 7]
    nn_i = np_.astype(jnp.int32)

    def cbody(i, s):
        sl = pl.ds(i * cc, cc)
        cio = _ciou_block(bp_ref[:, sl], bg_ref[:, sl])
        colid = lax.broadcasted_iota(jnp.int32, (1, cc), 1) + i * cc
        return s + jnp.sum(jnp.where(colid < nn_i, cio, 0.0))

    ciou_sum = lax.fori_loop(0, ntot // cc, cbody, jnp.float32(0.0))

    ce = ce_ref[...]
    n_neg_pos = jnp.minimum(negc, jnp.floor(3.0 * np_))
    kk = jnp.where(np_ > 0, n_neg_pos, jnp.minimum(negc, 100.0))

    def bbody(i, lohi):
        lo, hi = lohi
        mid = lo + (hi - lo) // 2
        tv = lax.bitcast_convert_type(jnp.full((1, 128), mid, jnp.int32),
                                      jnp.float32)
        t = jnp.max(tv)
        cnt = jnp.sum((ce >= t).astype(jnp.float32))
        ok = cnt >= kk
        return (jnp.where(ok, mid, lo), jnp.where(ok, hi, mid))

    lo, _hi = lax.fori_loop(0, 31, bbody,
                            (jnp.int32(0), jnp.int32(0x7F800000)))
    tv = lax.bitcast_convert_type(jnp.full((1, 128), lo, jnp.int32),
                                  jnp.float32)
    t = jnp.max(tv)
    cnt_gt = jnp.sum((ce > t).astype(jnp.float32))
    sum_gt = jnp.sum(jnp.where(ce > t, ce, 0.0))
    topk = jnp.where(kk >= 1.0, sum_gt + (kk - cnt_gt) * t, 0.0)
    loss_neg = jnp.where(kk >= negc, fneg, topk)
    nneg_div = jnp.where(np_ > 0, n_neg_pos, 100.0)
    loss_tr = (lpos + loss_neg) / (np_ + nneg_div)

    has_pos = np_ > 0
    sp = jnp.maximum(np_, 1.0)
    loss_tcl = jnp.where(
        has_pos, tclp / sp + 0.5 * tcln / jnp.maximum(ntot - np_, 1.0), 0.0)
    loss_rx = jnp.where(has_pos, rgx / (sp * 10.0), 0.0)
    loss_ry = jnp.where(has_pos, rgy / (sp * 10.0), 0.0)
    loss_bbox = jnp.where(has_pos, ciou_sum / sp, 0.0)

    vi = lax.broadcasted_iota(jnp.int32, (1, 128), 1)
    v = jnp.zeros((1, 128), jnp.float32)
    for i, val in enumerate([loss_tr, loss_tcl, loss_rx, loss_ry, loss_bbox]):
        v = jnp.where(vi == i, val, v)
    out_ref[...] = v


def _tc_finish(scal, ce2d, bp2d, bg2d, ntot):
    cc = 1280
    nr = ntot // 128
    return pl.pallas_call(
        functools.partial(_finish_kernel, ntot, cc),
        grid=(1,),
        in_specs=[pl.BlockSpec(memory_space=pltpu.SMEM),
                  pl.BlockSpec((nr, 128), lambda i: (0, 0)),
                  pl.BlockSpec((4, ntot), lambda i: (0, 0)),
                  pl.BlockSpec((4, ntot), lambda i: (0, 0))],
        out_specs=pl.BlockSpec((1, 128), lambda i: (0, 0)),
        out_shape=jax.ShapeDtypeStruct((1, 128), jnp.float32),
    )(scal, ce2d, bp2d, bg2d)


def _level(cls4d, reg4d, mask4d, map4d):
    bsz, _, h, w = cls4d.shape
    s = h * w
    n = bsz * s
    ck = 3200 if s % 3200 == 0 else s
    outs = _tc_stream(cls4d.reshape(bsz, 4, s), reg4d.reshape(bsz, 20, s),
                      mask4d.reshape(bsz, 3, s), map4d.reshape(bsz, 20, s),
                      ck)
    scal, ce, vals3d, rank3d = outs
    nn_vec = jnp.full((16,), scal[0, 0].astype(jnp.int32), jnp.int32)
    vals_cm = jnp.transpose(vals3d, (1, 0, 2)).reshape(-1)
    (boxes,) = _sc_scatter(rank3d.reshape(-1), vals_cm, nn_vec, n, ck)
    half = 4 * n + 2048
    res = _tc_finish(scal, ce.reshape(n // 128, 128),
                     boxes[:4 * n].reshape(4, n),
                     boxes[half:half + 4 * n].reshape(4, n), n)
    return res[0, :5]


def kernel(cls3, reg3, cls4, reg4, cls5, reg5,
           mask3, map3, mask4, map4, mask5, map5):
    tot = jnp.zeros((5,), jnp.float32)
    for (c, r, m, mp) in [(cls3, reg3, mask3, map3),
                          (cls4, reg4, mask4, map4),
                          (cls5, reg5, mask5, map5)]:
        tot = tot + _level(c, r, m, mp)
    return (tot[0], tot[1], tot[2], tot[3], tot[4])


# speedup vs baseline: 353.3885x; 1.0894x over previous
import functools
import math

import jax
import jax.numpy as jnp
from jax import lax
from jax.experimental import pallas as pl
from jax.experimental.pallas import tpu as pltpu
from jax.experimental.pallas import tpu_sc as plsc

_TRASH = 3.0e8


def _lane_cumsum(x):
    ck = x.shape[-1]
    it = lax.broadcasted_iota(jnp.int32, x.shape, 1)
    y = x
    sh = 1
    while sh < ck:
        y = y + jnp.where(it >= sh, pltpu.roll(y, sh, 1), 0.0)
        sh *= 2
    return y


def _stream_kernel(cls_ref, reg_ref, mask_ref, map_ref,
                   scal_ref, ce_ref, vals_ref, rank_ref, acc):
    b = pl.program_id(0)
    c = pl.program_id(1)
    nprog1 = pl.num_programs(1)
    g = b * nprog1 + c
    last = pl.num_programs(0) * nprog1 - 1

    @pl.when(g == 0)
    def _init():
        for i in range(9):
            acc[i] = 0.0

    cls_b = cls_ref[0]
    msk = mask_ref[0]
    reg_b = reg_ref[0]
    map_b = map_ref[0]

    tr_m = msk[0:1]
    tcl_m = msk[1:2]
    train_m = msk[2:3]

    def ce2(a, bb, t):
        m = jnp.maximum(a, bb)
        mn = jnp.minimum(a, bb)
        logz = m + jnp.log1p(jnp.exp(mn - m))
        return logz - jnp.where(t > 0.5, bb, a)

    ce_tr = ce2(cls_b[0:1], cls_b[1:2], tr_m)
    ce_tcl = ce2(cls_b[2:3], cls_b[3:4], tcl_m)

    pos = tr_m * train_m
    negm = (1.0 - tr_m) * train_m

    ce_ref[0] = jnp.where(negm > 0.5, ce_tr, -1.0)

    xp = reg_b[0:10]
    yp = reg_b[10:20]
    xg = map_b[0:10]
    yg = map_b[10:20]

    vals_ref[:, 0, 0] = jnp.concatenate(
        [jnp.min(xp, axis=0, keepdims=True),
         jnp.min(yp[0:5], axis=0, keepdims=True),
         jnp.max(xp, axis=0, keepdims=True),
         jnp.max(yp[5:10], axis=0, keepdims=True),
         jnp.min(xg, axis=0, keepdims=True),
         jnp.min(yg[0:5], axis=0, keepdims=True),
         jnp.max(xg, axis=0, keepdims=True),
         jnp.max(yg[5:10], axis=0, keepdims=True)], axis=0)

    cum = _lane_cumsum(pos)
    rank_f = acc[8] + cum - pos
    rank_ref[0] = jnp.where(pos > 0.5, rank_f, _TRASH).astype(jnp.int32)

    w = (tr_m + tcl_m) * 0.5
    dx = jnp.abs(xg - xp)
    slx = jnp.sum(jnp.where(dx < 1.0, 0.5 * dx * dx, dx - 0.5), axis=0,
                  keepdims=True)
    dy = jnp.abs(yg - yp)
    sly = jnp.sum(jnp.where(dy < 1.0, 0.5 * dy * dy, dy - 0.5), axis=0,
                  keepdims=True)

    npos_c = jnp.sum(pos)
    acc[0] = acc[0] + npos_c
    acc[1] = acc[1] + jnp.sum(negm)
    acc[2] = acc[2] + jnp.sum(pos * ce_tr)
    acc[3] = acc[3] + jnp.sum(negm * ce_tr)
    acc[4] = acc[4] + jnp.sum(pos * ce_tcl)
    acc[5] = acc[5] + jnp.sum((1.0 - pos) * ce_tcl)
    acc[6] = acc[6] + jnp.sum(pos * w * slx)
    acc[7] = acc[7] + jnp.sum(pos * w * sly)
    acc[8] = acc[8] + npos_c

    @pl.when(g == last)
    def _fin():
        vi = lax.broadcasted_iota(jnp.int32, (1, 128), 1)
        v = jnp.zeros((1, 128), jnp.float32)
        for i in range(8):
            v = jnp.where(vi == i, acc[i], v)
        scal_ref[...] = v


def _tc_stream(cls_l, reg_l, mask_l, map_l, ck):
    bsz, _, s = cls_l.shape
    nb = bsz * (s // ck)
    blk = lambda ch: pl.BlockSpec((1, ch, ck), lambda b, c: (b, 0, c))
    imap = lambda b, c, _s=(s // ck): (b * _s + c, 0, 0)
    outs = pl.pallas_call(
        _stream_kernel,
        grid=(bsz, s // ck),
        in_specs=[blk(4), blk(20), blk(3), blk(20)],
        out_specs=[pl.BlockSpec((1, 128), lambda b, c: (0, 0)),
                   pl.BlockSpec((1, 1, ck), imap),
                   pl.BlockSpec((8, 1, 1, ck),
                                lambda b, c, _s=(s // ck):
                                (0, b * _s + c, 0, 0)),
                   pl.BlockSpec((1, 1, ck), imap)],
        out_shape=[jax.ShapeDtypeStruct((1, 128), jnp.float32),
                   jax.ShapeDtypeStruct((nb, 1, ck), jnp.float32),
                   jax.ShapeDtypeStruct((8, nb, 1, ck), jnp.float32),
                   jax.ShapeDtypeStruct((nb, 1, ck), jnp.int32)],
        scratch_shapes=[pltpu.SMEM((16,), jnp.float32)],
    )(cls_l, reg_l, mask_l, map_l)
    return outs


def _sc_scatter_body(ntot, ck2, wsz, wpg,
                     rank_hbm, vals_hbm, nn_hbm, out_hbm,
                     nn_v, rk_v, v0, v1, v2, v3, i0, i1, i2, i3,
                     zbuf, spm):
    vv = (v0, v1, v2, v3)
    iv = (i0, i1, i2, i3)
    wn = ck2 // wsz
    gn = wn // wpg
    half = 4 * ntot + 2048
    osz = half // 16
    nfull = osz // 2048
    tail = (osz % 2048) // 128
    core = lax.axis_index("c")
    sid = lax.axis_index("s")

    z16 = jnp.zeros((16,), jnp.float32)
    for j in range(2048 // 16):
        zbuf[pl.ds(j * 16, 16)] = z16
    zbase = sid * osz

    def zc(i, c):
        pltpu.sync_copy(zbuf, spm.at[pl.ds(zbase + i * 2048, 2048)])
        return c

    lax.fori_loop(0, nfull, zc, 0)
    for t in range(tail):
        pltpu.sync_copy(
            zbuf.at[pl.ds(0, 128)],
            spm.at[pl.ds(zbase + nfull * 2048 + t * 128, 128)])

    pltpu.sync_copy(nn_hbm, nn_v)
    nn = nn_v[...]
    nt_v = jnp.full((16,), ntot, jnp.int32)
    lane = lax.iota(jnp.int32, 16)
    plsc.subcore_barrier()

    per_w = ntot // 16
    base = sid * per_w

    def chunk(ci, carry):
        off = base + ci * ck2
        pltpu.sync_copy(rank_hbm.at[pl.ds(off, ck2)], rk_v)
        for k in range(4):
            pltpu.sync_copy(
                vals_hbm.at[pl.ds((core * 4 + k) * ntot + off, ck2)], vv[k])

        def group(gi, carry2):
            for wl in range(wpg):
                w = gi * wpg + wl
                for j in range(wsz // 16):
                    r = rk_v[pl.ds(w * wsz + j * 16, 16)]
                    for k in range(4):
                        q = r + nn * k
                        idx = (q & 3) * ntot + (q >> 2)
                        trash = (4 * ntot + k * 128 + j * 16) + lane
                        iv[k][w, pl.ds(j * 16, 16)] = \
                            jnp.where(r < nt_v, idx, trash)
            for wl in range(wpg):
                w = gi * wpg + wl
                sl = pl.ds(w * wsz, wsz)
                for k in range(4):
                    pltpu.sync_copy(vv[k].at[sl], spm.at[iv[k].at[w]],
                                    add=True)
            return carry2

        lax.fori_loop(0, gn, group, 0)
        return carry

    lax.fori_loop(0, per_w // ck2, chunk, 0)

    plsc.subcore_barrier()
    pltpu.sync_copy(spm.at[pl.ds(sid * osz, osz)],
                    out_hbm.at[pl.ds(core * half + sid * osz, osz)])


def _sc_scatter(rank_flat, vals_flat, nn_vec, ntot, ck):
    per_w = ntot // 16
    ck2 = min(ck, per_w)
    wsz = 128 if ck2 % 128 == 0 else 80
    wn = ck2 // wsz
    wpg = 5
    mesh = plsc.VectorSubcoreMesh(core_axis_name="c", subcore_axis_name="s")
    fn = pl.kernel(
        functools.partial(_sc_scatter_body, ntot, ck2, wsz, wpg),
        mesh=mesh,
        out_type=[jax.ShapeDtypeStruct((2 * (4 * ntot + 2048),), jnp.float32)],
        scratch_types=[pltpu.VMEM((16,), jnp.int32),
                       pltpu.VMEM((ck2,), jnp.int32)]
                      + [pltpu.VMEM((ck2,), jnp.float32)] * 4
                      + [pltpu.VMEM((wn, wsz), jnp.int32)] * 4
                      + [pltpu.VMEM((2048,), jnp.float32),
                         pltpu.VMEM_SHARED((4 * ntot + 2048,), jnp.float32)],
    )
    return fn(rank_flat, vals_flat, nn_vec)


def _atan(x):
    t = jnp.abs(x)
    inv = t > 1.0
    z = jnp.where(inv, 1.0 / jnp.maximum(t, 1e-30), t)
    z2 = z * z
    p = jnp.float32(-0.0117212)
    p = p * z2 + 0.05265332
    p = p * z2 + -0.11643287
    p = p * z2 + 0.19354346
    p = p * z2 + -0.33262347
    p = p * z2 + 0.99997726
    p = p * z
    r = jnp.where(inv, jnp.float32(math.pi / 2) - p, p)
    return jnp.where(x < 0.0, -r, r)


def _ciou_block(P, G):
    eps = 1e-6
    px1, py1, px2, py2 = P[0:1], P[1:2], P[2:3], P[3:4]
    gx1, gy1, gx2, gy2 = G[0:1], G[1:2], G[2:3], G[3:4]
    wo = jnp.clip(jnp.minimum(px2, gx2) - jnp.maximum(px1, gx1), 0.0, None)
    ho = jnp.clip(jnp.minimum(py2, gy2) - jnp.maximum(py1, gy1), 0.0, None)
    overlap = wo * ho
    ap = (px2 - px1) * (py2 - py1)
    ag = (gx2 - gx1) * (gy2 - gy1)
    union = ap + ag - overlap + eps
    ious = overlap / union
    cw = jnp.clip(jnp.maximum(px2, gx2) - jnp.minimum(px1, gx1), 0.0, None)
    chh = jnp.clip(jnp.maximum(py2, gy2) - jnp.minimum(py1, gy1), 0.0, None)
    c2 = cw * cw + chh * chh + eps
    rho2 = ((gx1 + gx2) - (px1 + px2)) ** 2 / 4.0 \
        + ((gy1 + gy2) - (py1 + py2)) ** 2 / 4.0
    w1 = px2 - px1
    h1 = py2 - py1 + eps
    w2 = gx2 - gx1
    h2 = gy2 - gy1 + eps
    fct = 4.0 / (math.pi ** 2)
    v = fct * (_atan(w2 / h2) - _atan(w1 / h1)) ** 2
    alpha = (ious > 0.5).astype(jnp.float32) * v / (1.0 - ious + v)
    cious = ious - (rho2 / c2 + alpha * v)
    return 1.0 - jnp.clip(cious, -1.0, 1.0)


def _finish_kernel(ntot, cc, scal_ref, ce_ref, bp_ref, bg_ref, out_ref):
    np_ = scal_ref[0, 0]
    negc = scal_ref[0, 1]
    lpos = scal_ref[0, 2]
    fneg = scal_ref[0, 3]
    tclp = scal_ref[0, 4]
    tcln = scal_ref[0, 5]
    rgx = scal_ref[0, 6]
    rgy = scal_ref[0, 7]
    nn_i = np_.astype(jnp.int32)

    def cbody(i, s):
        sl = pl.ds(i * cc, cc)
        cio = _ciou_block(bp_ref[:, sl], bg_ref[:, sl])
        colid = lax.broadcasted_iota(jnp.int32, (1, cc), 1) + i * cc
        return s + jnp.sum(jnp.where(colid < nn_i, cio, 0.0))

    ciou_sum = lax.fori_loop(0, ntot // cc, cbody, jnp.float32(0.0))

    ce = ce_ref[...]
    n_neg_pos = jnp.minimum(negc, jnp.floor(3.0 * np_))
    kk = jnp.where(np_ > 0, n_neg_pos, jnp.minimum(negc, 100.0))

    def bbody(i, lohi):
        lo, hi = lohi
        mid = lo + (hi - lo) // 2
        tv = lax.bitcast_convert_type(jnp.full((1, 128), mid, jnp.int32),
                                      jnp.float32)
        t = jnp.max(tv)
        cnt = jnp.sum((ce >= t).astype(jnp.float32))
        ok = cnt >= kk
        return (jnp.where(ok, mid, lo), jnp.where(ok, hi, mid))

    lo, _hi = lax.fori_loop(0, 31, bbody,
                            (jnp.int32(0), jnp.int32(0x7F800000)))
    tv = lax.bitcast_convert_type(jnp.full((1, 128), lo, jnp.int32),
                                  jnp.float32)
    t = jnp.max(tv)
    cnt_gt = jnp.sum((ce > t).astype(jnp.float32))
    sum_gt = jnp.sum(jnp.where(ce > t, ce, 0.0))
    topk = jnp.where(kk >= 1.0, sum_gt + (kk - cnt_gt) * t, 0.0)
    loss_neg = jnp.where(kk >= negc, fneg, topk)
    nneg_div = jnp.where(np_ > 0, n_neg_pos, 100.0)
    loss_tr = (lpos + loss_neg) / (np_ + nneg_div)

    has_pos = np_ > 0
    sp = jnp.maximum(np_, 1.0)
    loss_tcl = jnp.where(
        has_pos, tclp / sp + 0.5 * tcln / jnp.maximum(ntot - np_, 1.0), 0.0)
    loss_rx = jnp.where(has_pos, rgx / (sp * 10.0), 0.0)
    loss_ry = jnp.where(has_pos, rgy / (sp * 10.0), 0.0)
    loss_bbox = jnp.where(has_pos, ciou_sum / sp, 0.0)

    vi = lax.broadcasted_iota(jnp.int32, (1, 128), 1)
    v = jnp.zeros((1, 128), jnp.float32)
    for i, val in enumerate([loss_tr, loss_tcl, loss_rx, loss_ry, loss_bbox]):
        v = jnp.where(vi == i, val, v)
    out_ref[...] = v


def _tc_finish(scal, ce2d, bp2d, bg2d, ntot):
    cc = 1280
    nr = ntot // 128
    return pl.pallas_call(
        functools.partial(_finish_kernel, ntot, cc),
        grid=(1,),
        in_specs=[pl.BlockSpec(memory_space=pltpu.SMEM),
                  pl.BlockSpec((nr, 128), lambda i: (0, 0)),
                  pl.BlockSpec((4, ntot), lambda i: (0, 0)),
                  pl.BlockSpec((4, ntot), lambda i: (0, 0))],
        out_specs=pl.BlockSpec((1, 128), lambda i: (0, 0)),
        out_shape=jax.ShapeDtypeStruct((1, 128), jnp.float32),
    )(scal, ce2d, bp2d, bg2d)


def _level(cls4d, reg4d, mask4d, map4d):
    bsz, _, h, w = cls4d.shape
    s = h * w
    n = bsz * s
    ck = 3200 if s % 3200 == 0 else s
    outs = _tc_stream(cls4d.reshape(bsz, 4, s), reg4d.reshape(bsz, 20, s),
                      mask4d.reshape(bsz, 3, s), map4d.reshape(bsz, 20, s),
                      ck)
    scal, ce, vals4d, rank3d = outs
    nn_vec = jnp.full((16,), scal[0, 0].astype(jnp.int32), jnp.int32)
    (boxes,) = _sc_scatter(rank3d.reshape(-1), vals4d.reshape(-1),
                           nn_vec, n, ck)
    half = 4 * n + 2048
    res = _tc_finish(scal, ce.reshape(n // 128, 128),
                     boxes[:4 * n].reshape(4, n),
                     boxes[half:half + 4 * n].reshape(4, n), n)
    return res[0, :5]


def kernel(cls3, reg3, cls4, reg4, cls5, reg5,
           mask3, map3, mask4, map4, mask5, map5):
    tot = jnp.zeros((5,), jnp.float32)
    for (c, r, m, mp) in [(cls3, reg3, mask3, map3),
                          (cls4, reg4, mask4, map4),
                          (cls5, reg5, mask5, map5)]:
        tot = tot + _level(c, r, m, mp)
    return (tot[0], tot[1], tot[2], tot[3], tot[4])
